# baseline (speedup 1.0000x reference)
import functools

import jax
import jax.numpy as jnp
from jax import lax
from jax.experimental import pallas as pl
from jax.experimental.pallas import tpu as pltpu
from jax.experimental.pallas import tpu_sc as plsc

BATCH = 16384
DIM = 64
VOCAB_P1 = 1_000_001
_SPLIT = 512_000

_NC, _NS = 2, 16
_NW = _NC * _NS
_BPW = BATCH // _NW
_ICH = 128
_NCHUNK = _BPW // _ICH

_TBLK = 2048


def _pack_body(ta_ref, tb_ref, eye_ref, out_ref):
    cat = jnp.concatenate([ta_ref[...], tb_ref[...]], axis=0)
    t = lax.dot_general(
        cat.astype(jnp.bfloat16), eye_ref[...], (((0,), (0,)), ((), ())),
        preferred_element_type=jnp.float32,
    )
    out_ref[...] = t


_LAST_B = (VOCAB_P1 - 1) // _TBLK


def _tc_pack(table_t, eye):
    grid = (_SPLIT // _TBLK,)
    return pl.pallas_call(
        _pack_body,
        grid=grid,
        in_specs=[
            pl.BlockSpec((DIM, _TBLK), lambda i: (0, i)),
            pl.BlockSpec(
                (DIM, _TBLK),
                lambda i: (0, jnp.minimum(i + _SPLIT // _TBLK, _LAST_B)),
            ),
            pl.BlockSpec((2 * DIM, 2 * DIM), lambda i: (0, 0)),
        ],
        out_specs=pl.BlockSpec((_TBLK, 2 * DIM), lambda i: (i, 0)),
        out_shape=jax.ShapeDtypeStruct((_SPLIT, 2 * DIM), jnp.float32),
    )(table_t, table_t, eye)


def _gather_body(pairs_hbm, pidx_hbm, out_hbm, idx_v, rows_v, sem):
    wid = lax.axis_index("s") * _NC + lax.axis_index("c")
    base = wid * _BPW
    pltpu.sync_copy(pidx_hbm.at[pl.ds(base, _BPW)], idx_v)

    def fire(g, carry):
        vals = idx_v[pl.ds(g * 16, 16)]
        for j in range(16):
            pltpu.async_copy(
                pairs_hbm.at[pl.ds(vals[j], 1)],
                rows_v.at[pl.ds(g * 16 + j, 1)],
                sem,
            )
        return carry

    lax.fori_loop(0, _BPW // 16, fire, 0)
    pltpu.make_async_copy(pairs_hbm.at[pl.ds(0, _BPW)], rows_v, sem).wait()
    pltpu.sync_copy(rows_v, out_hbm.at[pl.ds(base, _BPW)])


_sc_gather = functools.partial(
    pl.kernel,
    out_type=jax.ShapeDtypeStruct((BATCH, 2 * DIM), jnp.float32),
    mesh=plsc.VectorSubcoreMesh(core_axis_name="c", subcore_axis_name="s"),
    scratch_types=[
        pltpu.VMEM((_BPW,), jnp.int32),
        pltpu.VMEM((_BPW, 2 * DIM), jnp.float32),
        pltpu.SemaphoreType.DMA,
    ],
    compiler_params=pltpu.CompilerParams(use_tc_tiling_on_sc=True),
)(_gather_body)


def _fnn_body(pairs_ref, sel0_ref, sel1_ref, w1_ref, b1_ref, w2_ref, b2_ref,
              out_ref):
    emb = pairs_ref[:, :DIM] * sel0_ref[...] + pairs_ref[:, DIM:] * sel1_ref[...]
    h = jnp.dot(emb, w1_ref[...], preferred_element_type=jnp.float32)
    h = jnp.maximum(h + b1_ref[...], 0.0)
    out_ref[...] = (
        jnp.dot(h, w2_ref[...], preferred_element_type=jnp.float32) + b2_ref[...]
    )


_FNN_BLK = 2048


def _tc_fnn(pairs, sel0, sel1, w1, b1, w2, b2):
    grid = (BATCH // _FNN_BLK,)
    return pl.pallas_call(
        _fnn_body,
        grid=grid,
        in_specs=[
            pl.BlockSpec((_FNN_BLK, 2 * DIM), lambda i: (i, 0)),
            pl.BlockSpec((_FNN_BLK, 1), lambda i: (i, 0)),
            pl.BlockSpec((_FNN_BLK, 1), lambda i: (i, 0)),
            pl.BlockSpec((DIM, DIM), lambda i: (0, 0)),
            pl.BlockSpec((1, DIM), lambda i: (0, 0)),
            pl.BlockSpec((DIM, DIM), lambda i: (0, 0)),
            pl.BlockSpec((1, DIM), lambda i: (0, 0)),
        ],
        out_specs=pl.BlockSpec((_FNN_BLK, DIM), lambda i: (i, 0)),
        out_shape=jax.ShapeDtypeStruct((BATCH, DIM), jnp.float32),
    )(pairs, sel0, sel1, w1, b1, w2, b2)


def kernel(pk_idx, emb_table, W1, b1, W2, b2):
    idx = pk_idx.astype(jnp.int32)
    in_left = (idx < _SPLIT).astype(jnp.float32)[:, None]
    sel0 = in_left
    sel1 = 1.0 - in_left
    pidx = jnp.where(idx < _SPLIT, idx, idx - _SPLIT)
    eye = jnp.eye(2 * DIM, dtype=jnp.bfloat16)
    pairs_tbl = _tc_pack(emb_table.T, eye)
    pairs = _sc_gather(pairs_tbl, pidx)
    return _tc_fnn(
        pairs, sel0, sel1, W1, b1.reshape(1, DIM), W2, b2.reshape(1, DIM)
    )


# speedup vs baseline: 1.3464x; 1.3464x over previous
import functools

import jax
import jax.numpy as jnp
from jax import lax
from jax.experimental import pallas as pl
from jax.experimental.pallas import tpu as pltpu
from jax.experimental.pallas import tpu_sc as plsc

BATCH = 16384
DIM = 64
VOCAB_P1 = 1_000_001
_SPLIT = 512_000

_NC, _NS = 2, 16
_NW = _NC * _NS
_BPW = BATCH // _NW
_ICH = 128
_NCHUNK = _BPW // _ICH

_TBLK = 4096


def _pack_body(ta_ref, tb_ref, eye_ref, out_ref):
    cat = jnp.concatenate([ta_ref[...], tb_ref[...]], axis=0)
    t = lax.dot_general(
        cat.astype(jnp.bfloat16), eye_ref[...], (((0,), (0,)), ((), ())),
        preferred_element_type=jnp.float32,
    )
    out_ref[...] = t


_LAST_B = (VOCAB_P1 - 1) // _TBLK


def _tc_pack(table_t, eye):
    grid = (_SPLIT // _TBLK,)
    return pl.pallas_call(
        _pack_body,
        grid=grid,
        in_specs=[
            pl.BlockSpec((DIM, _TBLK), lambda i: (0, i)),
            pl.BlockSpec(
                (DIM, _TBLK),
                lambda i: (0, jnp.minimum(i + _SPLIT // _TBLK, _LAST_B)),
            ),
            pl.BlockSpec((2 * DIM, 2 * DIM), lambda i: (0, 0)),
        ],
        out_specs=pl.BlockSpec((_TBLK, 2 * DIM), lambda i: (i, 0)),
        out_shape=jax.ShapeDtypeStruct((_SPLIT, 2 * DIM), jnp.float32),
    )(table_t, table_t, eye)


def _gather_body(pairs_hbm, pidx_hbm, out_hbm, idx_v, rows_v, sem):
    wid = lax.axis_index("s") * _NC + lax.axis_index("c")
    base = wid * _BPW
    pltpu.sync_copy(pidx_hbm.at[pl.ds(base, _BPW)], idx_v)

    def fire(g, carry):
        vals = idx_v[pl.ds(g * 16, 16)]
        for j in range(16):
            pltpu.async_copy(
                pairs_hbm.at[pl.ds(vals[j], 1)],
                rows_v.at[pl.ds(g * 16 + j, 1)],
                sem,
            )
        return carry

    lax.fori_loop(0, _BPW // 16, fire, 0)
    pltpu.make_async_copy(pairs_hbm.at[pl.ds(0, _BPW)], rows_v, sem).wait()
    pltpu.sync_copy(rows_v, out_hbm.at[pl.ds(base, _BPW)])


_sc_gather = functools.partial(
    pl.kernel,
    out_type=jax.ShapeDtypeStruct((BATCH, 2 * DIM), jnp.float32),
    mesh=plsc.VectorSubcoreMesh(core_axis_name="c", subcore_axis_name="s"),
    scratch_types=[
        pltpu.VMEM((_BPW,), jnp.int32),
        pltpu.VMEM((_BPW, 2 * DIM), jnp.float32),
        pltpu.SemaphoreType.DMA,
    ],
    compiler_params=pltpu.CompilerParams(use_tc_tiling_on_sc=True),
)(_gather_body)


def _fnn_body(pairs_ref, sel0_ref, sel1_ref, w1_ref, b1_ref, w2_ref, b2_ref,
              out_ref):
    emb = pairs_ref[:, :DIM] * sel0_ref[...] + pairs_ref[:, DIM:] * sel1_ref[...]
    h = jnp.dot(emb, w1_ref[...], preferred_element_type=jnp.float32)
    h = jnp.maximum(h + b1_ref[...], 0.0)
    out_ref[...] = (
        jnp.dot(h, w2_ref[...], preferred_element_type=jnp.float32) + b2_ref[...]
    )


_FNN_BLK = 2048


def _tc_fnn(pairs, sel0, sel1, w1, b1, w2, b2):
    grid = (BATCH // _FNN_BLK,)
    return pl.pallas_call(
        _fnn_body,
        grid=grid,
        in_specs=[
            pl.BlockSpec((_FNN_BLK, 2 * DIM), lambda i: (i, 0)),
            pl.BlockSpec((_FNN_BLK, 1), lambda i: (i, 0)),
            pl.BlockSpec((_FNN_BLK, 1), lambda i: (i, 0)),
            pl.BlockSpec((DIM, DIM), lambda i: (0, 0)),
            pl.BlockSpec((1, DIM), lambda i: (0, 0)),
            pl.BlockSpec((DIM, DIM), lambda i: (0, 0)),
            pl.BlockSpec((1, DIM), lambda i: (0, 0)),
        ],
        out_specs=pl.BlockSpec((_FNN_BLK, DIM), lambda i: (i, 0)),
        out_shape=jax.ShapeDtypeStruct((BATCH, DIM), jnp.float32),
    )(pairs, sel0, sel1, w1, b1, w2, b2)


def kernel(pk_idx, emb_table, W1, b1, W2, b2):
    idx = pk_idx.astype(jnp.int32)
    in_left = (idx < _SPLIT).astype(jnp.float32)[:, None]
    sel0 = in_left
    sel1 = 1.0 - in_left
    pidx = jnp.where(idx < _SPLIT, idx, idx - _SPLIT)
    eye = jnp.eye(2 * DIM, dtype=jnp.bfloat16)
    pairs_tbl = _tc_pack(emb_table.T, eye)
    pairs = _sc_gather(pairs_tbl, pidx)
    return _tc_fnn(
        pairs, sel0, sel1, W1, b1.reshape(1, DIM), W2, b2.reshape(1, DIM)
    )


# speedup vs baseline: 1.5106x; 1.1220x over previous
import functools

import jax
import jax.numpy as jnp
from jax import lax
from jax.experimental import pallas as pl
from jax.experimental.pallas import tpu as pltpu
from jax.experimental.pallas import tpu_sc as plsc

BATCH = 16384
DIM = 64
VOCAB_P1 = 1_000_001
_SPLIT = 516_096

_NC, _NS = 2, 16
_NW = _NC * _NS
_BPW = BATCH // _NW
_ICH = 128
_NCHUNK = _BPW // _ICH

_TBLK = 8192


def _pack_body(ta_ref, tb_ref, eye_ref, out_ref):
    cat = jnp.concatenate([ta_ref[...], tb_ref[...]], axis=0)
    t = lax.dot_general(
        cat.astype(jnp.bfloat16), eye_ref[...], (((0,), (0,)), ((), ())),
        preferred_element_type=jnp.float32,
    )
    out_ref[...] = t


_LAST_B = (VOCAB_P1 - 1) // _TBLK


def _tc_pack(table_t, eye):
    grid = (_SPLIT // _TBLK,)
    return pl.pallas_call(
        _pack_body,
        grid=grid,
        in_specs=[
            pl.BlockSpec((DIM, _TBLK), lambda i: (0, i)),
            pl.BlockSpec(
                (DIM, _TBLK),
                lambda i: (0, jnp.minimum(i + _SPLIT // _TBLK, _LAST_B)),
            ),
            pl.BlockSpec((2 * DIM, 2 * DIM), lambda i: (0, 0)),
        ],
        out_specs=pl.BlockSpec((_TBLK, 2 * DIM), lambda i: (i, 0)),
        out_shape=jax.ShapeDtypeStruct((_SPLIT, 2 * DIM), jnp.float32),
    )(table_t, table_t, eye)


def _gather_body(pairs_hbm, pidx_hbm, out_hbm, idx_v, rows_v, sem):
    wid = lax.axis_index("s") * _NC + lax.axis_index("c")
    base = wid * _BPW
    pltpu.sync_copy(pidx_hbm.at[pl.ds(base, _BPW)], idx_v)

    def fire(g, carry):
        vals = idx_v[pl.ds(g * 16, 16)]
        for j in range(16):
            pltpu.async_copy(
                pairs_hbm.at[pl.ds(vals[j], 1)],
                rows_v.at[pl.ds(g * 16 + j, 1)],
                sem,
            )
        return carry

    lax.fori_loop(0, _BPW // 16, fire, 0)
    pltpu.make_async_copy(pairs_hbm.at[pl.ds(0, _BPW)], rows_v, sem).wait()
    pltpu.sync_copy(rows_v, out_hbm.at[pl.ds(base, _BPW)])


_sc_gather = functools.partial(
    pl.kernel,
    out_type=jax.ShapeDtypeStruct((BATCH, 2 * DIM), jnp.float32),
    mesh=plsc.VectorSubcoreMesh(core_axis_name="c", subcore_axis_name="s"),
    scratch_types=[
        pltpu.VMEM((_BPW,), jnp.int32),
        pltpu.VMEM((_BPW, 2 * DIM), jnp.float32),
        pltpu.SemaphoreType.DMA,
    ],
    compiler_params=pltpu.CompilerParams(use_tc_tiling_on_sc=True),
)(_gather_body)


def _fnn_body(pairs_ref, sel0_ref, sel1_ref, w1_ref, b1_ref, w2_ref, b2_ref,
              out_ref):
    emb = pairs_ref[:, :DIM] * sel0_ref[...] + pairs_ref[:, DIM:] * sel1_ref[...]
    h = jnp.dot(emb, w1_ref[...], preferred_element_type=jnp.float32)
    h = jnp.maximum(h + b1_ref[...], 0.0)
    out_ref[...] = (
        jnp.dot(h, w2_ref[...], preferred_element_type=jnp.float32) + b2_ref[...]
    )


_FNN_BLK = 2048


def _tc_fnn(pairs, sel0, sel1, w1, b1, w2, b2):
    grid = (BATCH // _FNN_BLK,)
    return pl.pallas_call(
        _fnn_body,
        grid=grid,
        in_specs=[
            pl.BlockSpec((_FNN_BLK, 2 * DIM), lambda i: (i, 0)),
            pl.BlockSpec((_FNN_BLK, 1), lambda i: (i, 0)),
            pl.BlockSpec((_FNN_BLK, 1), lambda i: (i, 0)),
            pl.BlockSpec((DIM, DIM), lambda i: (0, 0)),
            pl.BlockSpec((1, DIM), lambda i: (0, 0)),
            pl.BlockSpec((DIM, DIM), lambda i: (0, 0)),
            pl.BlockSpec((1, DIM), lambda i: (0, 0)),
        ],
        out_specs=pl.BlockSpec((_FNN_BLK, DIM), lambda i: (i, 0)),
        out_shape=jax.ShapeDtypeStruct((BATCH, DIM), jnp.float32),
    )(pairs, sel0, sel1, w1, b1, w2, b2)


def kernel(pk_idx, emb_table, W1, b1, W2, b2):
    idx = pk_idx.astype(jnp.int32)
    in_left = (idx < _SPLIT).astype(jnp.float32)[:, None]
    sel0 = in_left
    sel1 = 1.0 - in_left
    pidx = jnp.where(idx < _SPLIT, idx, idx - _SPLIT)
    eye = jnp.eye(2 * DIM, dtype=jnp.bfloat16)
    pairs_tbl = _tc_pack(emb_table.T, eye)
    pairs = _sc_gather(pairs_tbl, pidx)
    return _tc_fnn(
        pairs, sel0, sel1, W1, b1.reshape(1, DIM), W2, b2.reshape(1, DIM)
    )


# speedup vs baseline: 1.5576x; 1.0311x over previous
import functools

import jax
import jax.numpy as jnp
from jax import lax
from jax.experimental import pallas as pl
from jax.experimental.pallas import tpu as pltpu
from jax.experimental.pallas import tpu_sc as plsc

BATCH = 16384
DIM = 64
VOCAB_P1 = 1_000_001
_SPLIT = 507_904

_NC, _NS = 2, 16
_NW = _NC * _NS
_BPW = BATCH // _NW
_ICH = 128
_NCHUNK = _BPW // _ICH

_TBLK = 16384


def _pack_body(ta_ref, tb_ref, eye_ref, out_ref):
    cat = jnp.concatenate([ta_ref[...], tb_ref[...]], axis=0)
    t = lax.dot_general(
        cat.astype(jnp.bfloat16), eye_ref[...], (((0,), (0,)), ((), ())),
        preferred_element_type=jnp.float32,
    )
    out_ref[...] = t


_LAST_B = (VOCAB_P1 - 1) // _TBLK


def _tc_pack(table_t, eye):
    grid = (_SPLIT // _TBLK,)
    return pl.pallas_call(
        _pack_body,
        grid=grid,
        in_specs=[
            pl.BlockSpec((DIM, _TBLK), lambda i: (0, i)),
            pl.BlockSpec(
                (DIM, _TBLK),
                lambda i: (0, jnp.minimum(i + _SPLIT // _TBLK, _LAST_B)),
            ),
            pl.BlockSpec((2 * DIM, 2 * DIM), lambda i: (0, 0)),
        ],
        out_specs=pl.BlockSpec((_TBLK, 2 * DIM), lambda i: (i, 0)),
        out_shape=jax.ShapeDtypeStruct((_SPLIT, 2 * DIM), jnp.float32),
    )(table_t, table_t, eye)


def _gather_body(pairs_hbm, pidx_hbm, out_hbm, idx_v, rows_v, sem):
    wid = lax.axis_index("s") * _NC + lax.axis_index("c")
    base = wid * _BPW
    pltpu.sync_copy(pidx_hbm.at[pl.ds(base, _BPW)], idx_v)

    def fire(g, carry):
        vals = idx_v[pl.ds(g * 16, 16)]
        for j in range(16):
            pltpu.async_copy(
                pairs_hbm.at[pl.ds(vals[j], 1)],
                rows_v.at[pl.ds(g * 16 + j, 1)],
                sem,
            )
        return carry

    lax.fori_loop(0, _BPW // 16, fire, 0)
    pltpu.make_async_copy(pairs_hbm.at[pl.ds(0, _BPW)], rows_v, sem).wait()
    pltpu.sync_copy(rows_v, out_hbm.at[pl.ds(base, _BPW)])


_sc_gather = functools.partial(
    pl.kernel,
    out_type=jax.ShapeDtypeStruct((BATCH, 2 * DIM), jnp.float32),
    mesh=plsc.VectorSubcoreMesh(core_axis_name="c", subcore_axis_name="s"),
    scratch_types=[
        pltpu.VMEM((_BPW,), jnp.int32),
        pltpu.VMEM((_BPW, 2 * DIM), jnp.float32),
        pltpu.SemaphoreType.DMA,
    ],
    compiler_params=pltpu.CompilerParams(use_tc_tiling_on_sc=True),
)(_gather_body)


def _fnn_body(pairs_ref, sel0_ref, sel1_ref, w1_ref, b1_ref, w2_ref, b2_ref,
              out_ref):
    emb = pairs_ref[:, :DIM] * sel0_ref[...] + pairs_ref[:, DIM:] * sel1_ref[...]
    h = jnp.dot(emb, w1_ref[...], preferred_element_type=jnp.float32)
    h = jnp.maximum(h + b1_ref[...], 0.0)
    out_ref[...] = (
        jnp.dot(h, w2_ref[...], preferred_element_type=jnp.float32) + b2_ref[...]
    )


_FNN_BLK = 2048


def _tc_fnn(pairs, sel0, sel1, w1, b1, w2, b2):
    grid = (BATCH // _FNN_BLK,)
    return pl.pallas_call(
        _fnn_body,
        grid=grid,
        in_specs=[
            pl.BlockSpec((_FNN_BLK, 2 * DIM), lambda i: (i, 0)),
            pl.BlockSpec((_FNN_BLK, 1), lambda i: (i, 0)),
            pl.BlockSpec((_FNN_BLK, 1), lambda i: (i, 0)),
            pl.BlockSpec((DIM, DIM), lambda i: (0, 0)),
            pl.BlockSpec((1, DIM), lambda i: (0, 0)),
            pl.BlockSpec((DIM, DIM), lambda i: (0, 0)),
            pl.BlockSpec((1, DIM), lambda i: (0, 0)),
        ],
        out_specs=pl.BlockSpec((_FNN_BLK, DIM), lambda i: (i, 0)),
        out_shape=jax.ShapeDtypeStruct((BATCH, DIM), jnp.float32),
    )(pairs, sel0, sel1, w1, b1, w2, b2)


def kernel(pk_idx, emb_table, W1, b1, W2, b2):
    idx = pk_idx.astype(jnp.int32)
    in_left = (idx < _SPLIT).astype(jnp.float32)[:, None]
    sel0 = in_left
    sel1 = 1.0 - in_left
    pidx = jnp.where(idx < _SPLIT, idx, idx - _SPLIT)
    eye = jnp.eye(2 * DIM, dtype=jnp.bfloat16)
    pairs_tbl = _tc_pack(emb_table.T, eye)
    pairs = _sc_gather(pairs_tbl, pidx)
    return _tc_fnn(
        pairs, sel0, sel1, W1, b1.reshape(1, DIM), W2, b2.reshape(1, DIM)
    )
